# Initial kernel scaffold; baseline (speedup 1.0000x reference)
import functools

import jax
import jax.numpy as jnp
from jax import lax
from jax.experimental import pallas as pl
from jax.experimental.pallas import tpu as pltpu
from jax.experimental.pallas import tpu_sc as plsc

N = 10000
E = 320000
D_IN = 128
D_H = 64
NP = 10240
NC = 2
NS = 16
NW = NC * NS
EPW = E // NW
C = 80
NCH = EPW // C

_MESH = plsc.VectorSubcoreMesh(
    core_axis_name="c", subcore_axis_name="s", num_cores=NC, num_subcores=NS)


@functools.partial(
    pl.kernel,
    out_type=jax.ShapeDtypeStruct((NW, NP), jnp.float32),
    mesh=_MESH,
    scratch_types=[
        pltpu.VMEM((EPW,), jnp.int32),
        pltpu.VMEM((NP,), jnp.float32),
    ],
)
def _sc_degree(dst_hbm, out_hbm, dstv, acc):
    wid = lax.axis_index("c") * NS + lax.axis_index("s")
    pltpu.sync_copy(dst_hbm.at[wid], dstv)

    zeros16 = jnp.zeros((16,), jnp.float32)

    def zbody(i, c):
        acc[pl.ds(i * 16, 16)] = zeros16
        return c

    lax.fori_loop(0, NP // 16, zbody, 0)

    ones16 = jnp.ones((16,), jnp.float32)

    def body(i, c):
        idx = dstv[pl.ds(i * 16, 16)]
        plsc.addupdate_scatter(acc, [idx], ones16)
        return c

    lax.fori_loop(0, EPW // 16, body, 0)
    pltpu.sync_copy(acc, out_hbm.at[wid])


@functools.partial(
    pl.kernel,
    out_type=jax.ShapeDtypeStruct((NC, NP, D_H), jnp.float32),
    mesh=_MESH,
    scratch_types=[
        pltpu.VMEM((NCH, C), jnp.int32),
        pltpu.VMEM((NCH, C), jnp.int32),
        pltpu.VMEM((C, D_H), jnp.float32),
        pltpu.VMEM_SHARED((NP, D_H), jnp.float32),
    ],
)
def _sc_prop(q_hbm, src_hbm, dst_hbm, out_hbm, srcv, dstv, rows, acc_s):
    c = lax.axis_index("c")
    s = lax.axis_index("s")
    wid = c * NS + s
    pltpu.sync_copy(src_hbm.at[wid], srcv)
    pltpu.sync_copy(dst_hbm.at[wid], dstv)

    zeros16 = jnp.zeros((16,), jnp.float32)

    def zbody(i, cr):
        rows[i >> 2, pl.ds((i & 3) * 16, 16)] = zeros16
        return cr

    lax.fori_loop(0, C * D_H // 16, zbody, 0)
    rpw = NP // NS
    for k in range(rpw // C):
        pltpu.sync_copy(rows, acc_s.at[pl.ds(s * rpw + k * C, C), :])
    plsc.subcore_barrier()

    def body(j, cr):
        pltpu.sync_copy(q_hbm.at[srcv.at[j]], rows)
        pltpu.sync_copy(rows, acc_s.at[dstv.at[j]], add=True)
        return cr

    lax.fori_loop(0, NCH, body, 0)
    plsc.subcore_barrier()
    pltpu.sync_copy(acc_s.at[pl.ds(s * rpw, rpw), :],
                    out_hbm.at[c, pl.ds(s * rpw, rpw), :])


@functools.partial(
    pl.kernel,
    out_type=jax.ShapeDtypeStruct((NW, NP), jnp.float32),
    mesh=_MESH,
    scratch_types=[
        pltpu.VMEM((NP,), jnp.float32),
        pltpu.VMEM((EPW,), jnp.int32),
        pltpu.VMEM((EPW,), jnp.int32),
        pltpu.VMEM((NP,), jnp.float32),
    ],
)
def _sc_prop1(q_hbm, src_hbm, dst_hbm, out_hbm, qv, srcv, dstv, acc):
    wid = lax.axis_index("c") * NS + lax.axis_index("s")
    pltpu.sync_copy(q_hbm, qv)
    pltpu.sync_copy(src_hbm.at[wid], srcv)
    pltpu.sync_copy(dst_hbm.at[wid], dstv)

    zeros16 = jnp.zeros((16,), jnp.float32)

    def zbody(i, c):
        acc[pl.ds(i * 16, 16)] = zeros16
        return c

    lax.fori_loop(0, NP // 16, zbody, 0)

    def body(i, c):
        si = srcv[pl.ds(i * 16, 16)]
        di = dstv[pl.ds(i * 16, 16)]
        g = plsc.load_gather(qv, [si])
        plsc.addupdate_scatter(acc, [di], g)
        return c

    lax.fori_loop(0, EPW // 16, body, 0)
    pltpu.sync_copy(acc, out_hbm.at[wid])


_BR = 1280
_G = NP // _BR


def _tc_dinv_body(degp_ref, o_ref):
    deg = jnp.sum(degp_ref[...], axis=0) + 1.0
    o_ref[...] = lax.rsqrt(deg)


def _tc_dinv(deg_parts):
    return pl.pallas_call(
        _tc_dinv_body,
        grid=(_G,),
        in_specs=[pl.BlockSpec((NW, _BR, 1), lambda g: (0, g, 0))],
        out_specs=pl.BlockSpec((_BR, 1), lambda g: (g, 0)),
        out_shape=jax.ShapeDtypeStruct((NP, 1), jnp.float32),
    )(deg_parts.reshape(NW, NP, 1))


def _tc_l1_body(x_ref, w_ref, dinv_ref, o_ref):
    p = jnp.dot(x_ref[...], w_ref[...], preferred_element_type=jnp.float32)
    o_ref[...] = p * dinv_ref[...]


def _tc_l1(x, w1, dinv):
    return pl.pallas_call(
        _tc_l1_body,
        grid=(_G,),
        in_specs=[
            pl.BlockSpec((_BR, D_IN), lambda g: (g, 0)),
            pl.BlockSpec((D_IN, D_H), lambda g: (0, 0)),
            pl.BlockSpec((_BR, 1), lambda g: (g, 0)),
        ],
        out_specs=pl.BlockSpec((_BR, D_H), lambda g: (g, 0)),
        out_shape=jax.ShapeDtypeStruct((NP, D_H), jnp.float32),
    )(x, w1, dinv)


def _tc_mid_body(sp_ref, q_ref, dinv_ref, b_ref, w_ref, o_ref):
    s = sp_ref[0] + sp_ref[1]
    dinv = dinv_ref[...]
    h = jax.nn.relu(dinv * (s + q_ref[...]) + b_ref[...])
    o_ref[...] = jnp.dot(h, w_ref[...],
                         preferred_element_type=jnp.float32) * dinv


def _tc_mid(s_parts, q, dinv, b, w, d_out):
    return pl.pallas_call(
        _tc_mid_body,
        grid=(_G,),
        in_specs=[
            pl.BlockSpec((NC, _BR, D_H), lambda g: (0, g, 0)),
            pl.BlockSpec((_BR, D_H), lambda g: (g, 0)),
            pl.BlockSpec((_BR, 1), lambda g: (g, 0)),
            pl.BlockSpec((1, D_H), lambda g: (0, 0)),
            pl.BlockSpec((D_H, d_out), lambda g: (0, 0)),
        ],
        out_specs=pl.BlockSpec((_BR, d_out), lambda g: (g, 0)),
        out_shape=jax.ShapeDtypeStruct((NP, d_out), jnp.float32),
    )(s_parts, q, dinv, b.reshape(1, D_H), w)


def _tc_final_body(sp_ref, q_ref, dinv_ref, b_ref, o_ref):
    s = jnp.sum(sp_ref[...], axis=0)
    o_ref[...] = dinv_ref[...] * (s + q_ref[...]) + b_ref[0, 0]


def _tc_final(s_parts, q3, dinv, b3):
    return pl.pallas_call(
        _tc_final_body,
        grid=(_G,),
        in_specs=[
            pl.BlockSpec((NW, _BR, 1), lambda g: (0, g, 0)),
            pl.BlockSpec((_BR, 1), lambda g: (g, 0)),
            pl.BlockSpec((_BR, 1), lambda g: (g, 0)),
            pl.BlockSpec((1, 1), lambda g: (0, 0)),
        ],
        out_specs=pl.BlockSpec((_BR, 1), lambda g: (g, 0)),
        out_shape=jax.ShapeDtypeStruct((NP, 1), jnp.float32),
    )(s_parts.reshape(NW, NP, 1), q3, dinv, b3.reshape(1, 1))


def kernel(x, edge_index, W1, b1, W2, b2, W3, b3):
    src = edge_index[0]
    dst = edge_index[1]
    src3 = src.reshape(NW, NCH, C)
    dst3 = dst.reshape(NW, NCH, C)
    src2 = src.reshape(NW, EPW)
    dst2 = dst.reshape(NW, EPW)
    x_pad = jnp.pad(x, ((0, NP - N), (0, 0)))

    deg_parts = _sc_degree(dst2)
    dinv = _tc_dinv(deg_parts)

    q1 = _tc_l1(x_pad, W1, dinv)
    s1 = _sc_prop(q1, src3, dst3)
    q2 = _tc_mid(s1, q1, dinv, b1, W2, D_H)
    s2 = _sc_prop(s2_in := q2, src3, dst3)
    q3 = _tc_mid(s2, q2, dinv, b2, W3, 1)
    s3 = _sc_prop1(q3.reshape(NP), src2, dst2)
    out = _tc_final(s3, q3, dinv, b3)
    return out[:N]


# speedup vs baseline: 19.0534x; 19.0534x over previous
import functools

import jax
import jax.numpy as jnp
from jax import lax
from jax.experimental import pallas as pl
from jax.experimental.pallas import tpu as pltpu
from jax.experimental.pallas import tpu_sc as plsc

N = 10000
E = 320000
D_IN = 128
D_H = 64
NP = 10240
NC = 2
NS = 16
NW = NC * NS
EPW = E // NW
C = 80
NCH = EPW // C

_MESH = plsc.VectorSubcoreMesh(
    core_axis_name="c", subcore_axis_name="s", num_cores=NC, num_subcores=NS)


@functools.partial(
    pl.kernel,
    out_type=jax.ShapeDtypeStruct((NW, NP), jnp.float32),
    mesh=_MESH,
    scratch_types=[
        pltpu.VMEM((EPW,), jnp.int32),
        pltpu.VMEM((NP,), jnp.float32),
    ],
    compiler_params=pltpu.CompilerParams(needs_layout_passes=False),
)
def _sc_degree(dst_hbm, out_hbm, dstv, acc):
    wid = lax.axis_index("c") * NS + lax.axis_index("s")
    pltpu.sync_copy(dst_hbm.at[wid], dstv)

    zeros16 = jnp.zeros((16,), jnp.float32)

    def zbody(i, c):
        acc[pl.ds(i * 16, 16)] = zeros16
        return c

    lax.fori_loop(0, NP // 16, zbody, 0)

    ones16 = jnp.ones((16,), jnp.float32)

    def body(i, c):
        idx = dstv[pl.ds(i * 16, 16)]
        plsc.addupdate_scatter(acc, [idx], ones16)
        return c

    lax.fori_loop(0, EPW // 16, body, 0)
    pltpu.sync_copy(acc, out_hbm.at[wid])


@functools.partial(
    pl.kernel,
    out_type=jax.ShapeDtypeStruct((NC, NP, D_H), jnp.float32),
    mesh=_MESH,
    scratch_types=[
        pltpu.VMEM((NCH, C), jnp.int32),
        pltpu.VMEM((NCH, C), jnp.int32),
        pltpu.VMEM((C, D_H), jnp.float32),
        pltpu.VMEM_SHARED((NP, D_H), jnp.float32),
    ],
    compiler_params=pltpu.CompilerParams(use_tc_tiling_on_sc=False),
)
def _sc_prop(q_hbm, src_hbm, dst_hbm, out_hbm, srcv, dstv, rows, acc_s):
    c = lax.axis_index("c")
    s = lax.axis_index("s")
    wid = c * NS + s
    pltpu.sync_copy(src_hbm.at[wid], srcv)
    pltpu.sync_copy(dst_hbm.at[wid], dstv)

    zeros16 = jnp.zeros((16,), jnp.float32)

    def zbody(i, cr):
        rows[i >> 2, pl.ds((i & 3) * 16, 16)] = zeros16
        return cr

    lax.fori_loop(0, C * D_H // 16, zbody, 0)
    rpw = NP // NS
    for k in range(rpw // C):
        pltpu.sync_copy(rows, acc_s.at[pl.ds(s * rpw + k * C, C), :])
    plsc.subcore_barrier()

    def body(j, cr):
        pltpu.sync_copy(q_hbm.at[srcv.at[j]], rows)
        pltpu.sync_copy(rows, acc_s.at[dstv.at[j]], add=True)
        return cr

    lax.fori_loop(0, NCH, body, 0)
    plsc.subcore_barrier()
    pltpu.sync_copy(acc_s.at[pl.ds(s * rpw, rpw), :],
                    out_hbm.at[c, pl.ds(s * rpw, rpw), :])


@functools.partial(
    pl.kernel,
    out_type=jax.ShapeDtypeStruct((NW, NP), jnp.float32),
    mesh=_MESH,
    scratch_types=[
        pltpu.VMEM((NP,), jnp.float32),
        pltpu.VMEM((EPW,), jnp.int32),
        pltpu.VMEM((EPW,), jnp.int32),
        pltpu.VMEM((NP,), jnp.float32),
    ],
    compiler_params=pltpu.CompilerParams(needs_layout_passes=False),
)
def _sc_prop1(q_hbm, src_hbm, dst_hbm, out_hbm, qv, srcv, dstv, acc):
    wid = lax.axis_index("c") * NS + lax.axis_index("s")
    pltpu.sync_copy(q_hbm, qv)
    pltpu.sync_copy(src_hbm.at[wid], srcv)
    pltpu.sync_copy(dst_hbm.at[wid], dstv)

    zeros16 = jnp.zeros((16,), jnp.float32)

    def zbody(i, c):
        acc[pl.ds(i * 16, 16)] = zeros16
        return c

    lax.fori_loop(0, NP // 16, zbody, 0)

    def body(i, c):
        si = srcv[pl.ds(i * 16, 16)]
        di = dstv[pl.ds(i * 16, 16)]
        g = plsc.load_gather(qv, [si])
        plsc.addupdate_scatter(acc, [di], g)
        return c

    lax.fori_loop(0, EPW // 16, body, 0)
    pltpu.sync_copy(acc, out_hbm.at[wid])


_BR = 1280
_G = NP // _BR


def _tc_dinv_body(degp_ref, o_ref):
    deg = jnp.sum(degp_ref[...], axis=0) + 1.0
    o_ref[...] = lax.rsqrt(deg)


def _tc_dinv(deg_parts):
    return pl.pallas_call(
        _tc_dinv_body,
        grid=(_G,),
        in_specs=[pl.BlockSpec((NW, _BR, 1), lambda g: (0, g, 0))],
        out_specs=pl.BlockSpec((_BR, 1), lambda g: (g, 0)),
        out_shape=jax.ShapeDtypeStruct((NP, 1), jnp.float32),
    )(deg_parts.reshape(NW, NP, 1))


def _tc_l1_body(x_ref, w_ref, dinv_ref, o_ref):
    p = jnp.dot(x_ref[...], w_ref[...], preferred_element_type=jnp.float32)
    o_ref[...] = p * dinv_ref[...]


def _tc_l1(x, w1, dinv):
    return pl.pallas_call(
        _tc_l1_body,
        grid=(_G,),
        in_specs=[
            pl.BlockSpec((_BR, D_IN), lambda g: (g, 0)),
            pl.BlockSpec((D_IN, D_H), lambda g: (0, 0)),
            pl.BlockSpec((_BR, 1), lambda g: (g, 0)),
        ],
        out_specs=pl.BlockSpec((_BR, D_H), lambda g: (g, 0)),
        out_shape=jax.ShapeDtypeStruct((NP, D_H), jnp.float32),
    )(x, w1, dinv)


def _tc_mid_body(sp_ref, q_ref, dinv_ref, b_ref, w_ref, o_ref):
    s = sp_ref[0] + sp_ref[1]
    dinv = dinv_ref[...]
    h = jax.nn.relu(dinv * (s + q_ref[...]) + b_ref[...])
    o_ref[...] = jnp.dot(h, w_ref[...],
                         preferred_element_type=jnp.float32) * dinv


def _tc_mid(s_parts, q, dinv, b, w, d_out):
    return pl.pallas_call(
        _tc_mid_body,
        grid=(_G,),
        in_specs=[
            pl.BlockSpec((NC, _BR, D_H), lambda g: (0, g, 0)),
            pl.BlockSpec((_BR, D_H), lambda g: (g, 0)),
            pl.BlockSpec((_BR, 1), lambda g: (g, 0)),
            pl.BlockSpec((1, D_H), lambda g: (0, 0)),
            pl.BlockSpec((D_H, d_out), lambda g: (0, 0)),
        ],
        out_specs=pl.BlockSpec((_BR, d_out), lambda g: (g, 0)),
        out_shape=jax.ShapeDtypeStruct((NP, d_out), jnp.float32),
    )(s_parts, q, dinv, b.reshape(1, D_H), w)


def _tc_final_body(sp_ref, q_ref, dinv_ref, b_ref, o_ref):
    s = jnp.sum(sp_ref[...], axis=0)
    o_ref[...] = dinv_ref[...] * (s + q_ref[...]) + b_ref[0, 0]


def _tc_final(s_parts, q3, dinv, b3):
    return pl.pallas_call(
        _tc_final_body,
        grid=(_G,),
        in_specs=[
            pl.BlockSpec((NW, _BR, 1), lambda g: (0, g, 0)),
            pl.BlockSpec((_BR, 1), lambda g: (g, 0)),
            pl.BlockSpec((_BR, 1), lambda g: (g, 0)),
            pl.BlockSpec((1, 1), lambda g: (0, 0)),
        ],
        out_specs=pl.BlockSpec((_BR, 1), lambda g: (g, 0)),
        out_shape=jax.ShapeDtypeStruct((NP, 1), jnp.float32),
    )(s_parts.reshape(NW, NP, 1), q3, dinv, b3.reshape(1, 1))


def kernel(x, edge_index, W1, b1, W2, b2, W3, b3):
    src = edge_index[0]
    dst = edge_index[1]
    src3 = src.reshape(NW, NCH, C)
    dst3 = dst.reshape(NW, NCH, C)
    src2 = src.reshape(NW, EPW)
    dst2 = dst.reshape(NW, EPW)
    x_pad = jnp.pad(x, ((0, NP - N), (0, 0)))

    deg_parts = _sc_degree(dst2)
    dinv = _tc_dinv(deg_parts)

    q1 = _tc_l1(x_pad, W1, dinv)
    s1 = _sc_prop(q1, src3, dst3)
    q2 = _tc_mid(s1, q1, dinv, b1, W2, D_H)
    s2 = _sc_prop(q2, src3, dst3)
    q3 = _tc_mid(s2, q2, dinv, b2, W3, 1)
    s3 = _sc_prop1(q3.reshape(NP), src2, dst2)
    out = _tc_final(s3, q3, dinv, b3)
    return out[:N]


# speedup vs baseline: 35.9675x; 1.8877x over previous
import functools

import jax
import jax.numpy as jnp
from jax import lax
from jax.experimental import pallas as pl
from jax.experimental.pallas import tpu as pltpu
from jax.experimental.pallas import tpu_sc as plsc

N = 10000
E = 320000
D_IN = 128
D_H = 64
NP = 10240
NPK = NP // 2
NC = 2
NS = 16
NW = NC * NS
EPW = E // NW
C = 80
NCH = EPW // C
NCH2 = NCH + 1

_MESH = plsc.VectorSubcoreMesh(
    core_axis_name="c", subcore_axis_name="s", num_cores=NC, num_subcores=NS)


@functools.partial(
    pl.kernel,
    out_type=jax.ShapeDtypeStruct((NW, NP), jnp.float32),
    mesh=_MESH,
    scratch_types=[
        pltpu.VMEM((EPW,), jnp.int32),
        pltpu.VMEM((NP,), jnp.float32),
    ],
    compiler_params=pltpu.CompilerParams(needs_layout_passes=False),
)
def _sc_degree(dst_hbm, out_hbm, dstv, acc):
    wid = lax.axis_index("c") * NS + lax.axis_index("s")
    pltpu.sync_copy(dst_hbm.at[wid], dstv)

    zeros16 = jnp.zeros((16,), jnp.float32)

    def zbody(i, c):
        acc[pl.ds(i * 16, 16)] = zeros16
        return c

    lax.fori_loop(0, NP // 16, zbody, 0)

    ones16 = jnp.ones((16,), jnp.float32)

    def body(i, c):
        idx = dstv[pl.ds(i * 16, 16)]
        plsc.addupdate_scatter(acc, [idx], ones16)
        return c

    lax.fori_loop(0, EPW // 16, body, 0)
    pltpu.sync_copy(acc, out_hbm.at[wid])


@functools.partial(
    pl.kernel,
    out_type=jax.ShapeDtypeStruct((NC, NP, D_H), jnp.float32),
    mesh=_MESH,
    scratch_types=[
        pltpu.VMEM((NCH2, C), jnp.int32),
        pltpu.VMEM((NCH2, C), jnp.int32),
        pltpu.VMEM((C, D_H), jnp.float32),
        pltpu.VMEM((C, D_H), jnp.float32),
        pltpu.VMEM_SHARED((NP, D_H), jnp.float32),
        pltpu.SemaphoreType.DMA,
    ],
    compiler_params=pltpu.CompilerParams(use_tc_tiling_on_sc=False),
)
def _sc_prop(q_hbm, src_hbm, dst_hbm, out_hbm, srcv, dstv, rows0, rows1,
             acc_s, sem):
    c = lax.axis_index("c")
    s = lax.axis_index("s")
    wid = c * NS + s
    pltpu.sync_copy(src_hbm.at[wid], srcv)
    pltpu.sync_copy(dst_hbm.at[wid], dstv)

    zeros16 = jnp.zeros((16,), jnp.float32)

    def zbody(i, cr):
        rows0[i >> 2, pl.ds((i & 3) * 16, 16)] = zeros16
        return cr

    lax.fori_loop(0, C * D_H // 16, zbody, 0)
    rpw = NP // NS
    for k in range(rpw // C):
        pltpu.sync_copy(rows0, acc_s.at[pl.ds(s * rpw + k * C, C), :])

    pltpu.async_copy(q_hbm.at[srcv.at[0]], rows0, sem)
    pltpu.async_copy(q_hbm.at[srcv.at[1]], rows1, sem)
    plsc.subcore_barrier()

    def body(k, cr):
        j0 = 2 * k
        pltpu.make_async_copy(q_hbm.at[srcv.at[j0]], rows0, sem).wait()
        pltpu.sync_copy(rows0, acc_s.at[dstv.at[j0]], add=True)

        @pl.when(j0 + 2 < NCH2)
        def _():
            pltpu.async_copy(q_hbm.at[srcv.at[j0 + 2]], rows0, sem)

        pltpu.make_async_copy(q_hbm.at[srcv.at[j0 + 1]], rows1, sem).wait()
        pltpu.sync_copy(rows1, acc_s.at[dstv.at[j0 + 1]], add=True)

        @pl.when(j0 + 3 < NCH2)
        def _():
            pltpu.async_copy(q_hbm.at[srcv.at[j0 + 3]], rows1, sem)

        return cr

    lax.fori_loop(0, NCH2 // 2, body, 0)
    plsc.subcore_barrier()
    pltpu.sync_copy(acc_s.at[pl.ds(s * rpw, rpw), :],
                    out_hbm.at[c, pl.ds(s * rpw, rpw), :])


@functools.partial(
    pl.kernel,
    out_type=jax.ShapeDtypeStruct((NW, NP), jnp.float32),
    mesh=_MESH,
    scratch_types=[
        pltpu.VMEM((NP,), jnp.float32),
        pltpu.VMEM((EPW,), jnp.int32),
        pltpu.VMEM((EPW,), jnp.int32),
        pltpu.VMEM((NP,), jnp.float32),
    ],
    compiler_params=pltpu.CompilerParams(needs_layout_passes=False),
)
def _sc_prop1(q_hbm, src_hbm, dst_hbm, out_hbm, qv, srcv, dstv, acc):
    wid = lax.axis_index("c") * NS + lax.axis_index("s")
    pltpu.sync_copy(q_hbm, qv)
    pltpu.sync_copy(src_hbm.at[wid], srcv)
    pltpu.sync_copy(dst_hbm.at[wid], dstv)

    zeros16 = jnp.zeros((16,), jnp.float32)

    def zbody(i, c):
        acc[pl.ds(i * 16, 16)] = zeros16
        return c

    lax.fori_loop(0, NP // 16, zbody, 0)

    def body(i, c):
        si = srcv[pl.ds(i * 16, 16)]
        di = dstv[pl.ds(i * 16, 16)]
        g = plsc.load_gather(qv, [si])
        plsc.addupdate_scatter(acc, [di], g)
        return c

    lax.fori_loop(0, EPW // 16, body, 0)
    pltpu.sync_copy(acc, out_hbm.at[wid])


_BRK = 1024
_G = NPK // _BRK


def _tc_dinv_body(degp_ref, o_ref):
    deg = jnp.sum(degp_ref[...], axis=0) + 1.0
    o_ref[...] = lax.rsqrt(deg).reshape(1, NP)


def _tc_dinv(deg_parts):
    return pl.pallas_call(
        _tc_dinv_body,
        out_shape=jax.ShapeDtypeStruct((1, NP), jnp.float32),
    )(deg_parts)


def _tc_l1_body(x_ref, w_ref, dinv_ref, o_ref):
    p = jnp.dot(x_ref[...], w_ref[...], preferred_element_type=jnp.float32)
    o_ref[...] = p * dinv_ref[...]


def _tc_l1(x2, w1blk, dinv_pk):
    return pl.pallas_call(
        _tc_l1_body,
        grid=(_G,),
        in_specs=[
            pl.BlockSpec((_BRK, 2 * D_IN), lambda g: (g, 0)),
            pl.BlockSpec((2 * D_IN, 128), lambda g: (0, 0)),
            pl.BlockSpec((_BRK, 128), lambda g: (g, 0)),
        ],
        out_specs=pl.BlockSpec((_BRK, 128), lambda g: (g, 0)),
        out_shape=jax.ShapeDtypeStruct((NPK, 128), jnp.float32),
    )(x2, w1blk, dinv_pk)


def _tc_mid_body(sp_ref, q_ref, dinv_ref, b_ref, w_ref, o_ref):
    s = sp_ref[0] + sp_ref[1]
    dinv = dinv_ref[...]
    h = jax.nn.relu(dinv * (s + q_ref[...]) + b_ref[...])
    o_ref[...] = jnp.dot(h, w_ref[...],
                         preferred_element_type=jnp.float32) * dinv


def _tc_mid(s_parts_pk, q_pk, dinv_pk, b_pk, wblk):
    return pl.pallas_call(
        _tc_mid_body,
        grid=(_G,),
        in_specs=[
            pl.BlockSpec((NC, _BRK, 128), lambda g: (0, g, 0)),
            pl.BlockSpec((_BRK, 128), lambda g: (g, 0)),
            pl.BlockSpec((_BRK, 128), lambda g: (g, 0)),
            pl.BlockSpec((1, 128), lambda g: (0, 0)),
            pl.BlockSpec((128, 128), lambda g: (0, 0)),
        ],
        out_specs=pl.BlockSpec((_BRK, 128), lambda g: (g, 0)),
        out_shape=jax.ShapeDtypeStruct((NPK, 128), jnp.float32),
    )(s_parts_pk, q_pk, dinv_pk, b_pk, wblk)


def _tc_final_body(sp_ref, q_ref, dinv_ref, b_ref, o_ref):
    s = jnp.sum(sp_ref[...], axis=0)
    o_ref[...] = dinv_ref[...] * (s.reshape(1, NP) + q_ref[...]) + b_ref[0, 0]


def _tc_final(s_parts, q3f, dinv1d, b3):
    return pl.pallas_call(
        _tc_final_body,
        out_shape=jax.ShapeDtypeStruct((1, NP), jnp.float32),
    )(s_parts, q3f, dinv1d, b3.reshape(1, 1))


def kernel(x, edge_index, W1, b1, W2, b2, W3, b3):
    src = edge_index[0]
    dst = edge_index[1]
    i32 = jnp.int32
    src3 = jnp.concatenate(
        [src.reshape(NW, NCH, C), jnp.zeros((NW, 1, C), i32)], axis=1)
    dst3 = jnp.concatenate(
        [dst.reshape(NW, NCH, C), jnp.full((NW, 1, C), NP - 1, i32)], axis=1)
    src2 = src.reshape(NW, EPW)
    dst2 = dst.reshape(NW, EPW)

    x_pad = jnp.pad(x, ((0, NP - N), (0, 0)))
    x2 = x_pad.reshape(NPK, 2 * D_IN)
    w1blk = (jnp.zeros((2 * D_IN, 128), jnp.float32)
             .at[:D_IN, :D_H].set(W1).at[D_IN:, D_H:].set(W1))
    w2blk = (jnp.zeros((128, 128), jnp.float32)
             .at[:D_H, :D_H].set(W2).at[D_H:, D_H:].set(W2))
    w3blk = (jnp.zeros((128, 128), jnp.float32)
             .at[:D_H, 0:1].set(W3).at[D_H:, D_H:D_H + 1].set(W3))
    b1pk = jnp.concatenate([b1, b1]).reshape(1, 128)
    b2pk = jnp.concatenate([b2, b2]).reshape(1, 128)

    deg_parts = _sc_degree(dst2)
    dinv1d = _tc_dinv(deg_parts)
    dinv_pk = jnp.repeat(dinv1d.reshape(NP), D_H).reshape(NPK, 128)

    q1 = _tc_l1(x2, w1blk, dinv_pk)
    s1 = _sc_prop(q1.reshape(NP, D_H), src3, dst3)
    q2 = _tc_mid(s1.reshape(NC, NPK, 128), q1, dinv_pk, b1pk, w2blk)
    s2 = _sc_prop(q2.reshape(NP, D_H), src3, dst3)
    q3 = _tc_mid(s2.reshape(NC, NPK, 128), q2, dinv_pk, b2pk, w3blk)
    q3f = q3.reshape(NPK, 2, D_H)[:, :, 0].reshape(NP)
    s3 = _sc_prop1(q3f, src2, dst2)
    out = _tc_final(s3, q3f.reshape(1, NP), dinv1d, b3)
    return out[0, :N].reshape(N, 1)


# speedup vs baseline: 49.2260x; 1.3686x over previous
import functools

import jax
import jax.numpy as jnp
from jax import lax
from jax.experimental import pallas as pl
from jax.experimental.pallas import tpu as pltpu
from jax.experimental.pallas import tpu_sc as plsc

N = 10000
E = 320000
D_IN = 128
D_H = 64
NP = 10240
NPK = NP // 2
NC = 2
NS = 16
NW = NC * NS
EPW = E // NW
C = 80
NCH = EPW // C

_MESH = plsc.VectorSubcoreMesh(
    core_axis_name="c", subcore_axis_name="s", num_cores=NC, num_subcores=NS)


@functools.partial(
    pl.kernel,
    out_type=jax.ShapeDtypeStruct((NW, NP), jnp.float32),
    mesh=_MESH,
    scratch_types=[
        pltpu.VMEM((EPW,), jnp.int32),
        pltpu.VMEM((NP,), jnp.float32),
    ],
    compiler_params=pltpu.CompilerParams(
        needs_layout_passes=False, use_tc_tiling_on_sc=False),
)
def _sc_degree(dst_hbm, out_hbm, dstv, acc):
    wid = lax.axis_index("c") * NS + lax.axis_index("s")
    pltpu.sync_copy(dst_hbm.at[wid], dstv)

    zeros16 = jnp.zeros((16,), jnp.float32)

    def zbody(i, c):
        acc[pl.ds(i * 16, 16)] = zeros16
        return c

    lax.fori_loop(0, NP // 16, zbody, 0)

    ones16 = jnp.ones((16,), jnp.float32)

    def body(i, c):
        idx = dstv[pl.ds(i * 16, 16)]
        plsc.addupdate_scatter(acc, [idx], ones16)
        return c

    lax.fori_loop(0, EPW // 16, body, 0)
    pltpu.sync_copy(acc, out_hbm.at[wid])


@functools.partial(
    pl.kernel,
    out_type=jax.ShapeDtypeStruct((NC, NP, D_H), jnp.float32),
    mesh=_MESH,
    scratch_types=[
        pltpu.VMEM((NCH, C), jnp.int32),
        pltpu.VMEM((NCH, C), jnp.int32),
        pltpu.VMEM((C, D_H), jnp.float32),
        pltpu.VMEM((C, D_H), jnp.float32),
        pltpu.VMEM_SHARED((NP, D_H), jnp.float32),
        pltpu.SemaphoreType.DMA,
    ],
    compiler_params=pltpu.CompilerParams(use_tc_tiling_on_sc=False),
)
def _sc_prop(q_hbm, src_hbm, dst_hbm, out_hbm, srcv, dstv, rows0, rows1,
             acc_s, sem):
    c = lax.axis_index("c")
    s = lax.axis_index("s")
    wid = c * NS + s
    pltpu.sync_copy(src_hbm.at[wid], srcv)
    pltpu.sync_copy(dst_hbm.at[wid], dstv)

    zeros16 = jnp.zeros((16,), jnp.float32)

    def zbody(i, cr):
        rows0[i >> 2, pl.ds((i & 3) * 16, 16)] = zeros16
        return cr

    lax.fori_loop(0, C * D_H // 16, zbody, 0)
    rpw = NP // NS
    for k in range(rpw // C):
        pltpu.sync_copy(rows0, acc_s.at[pl.ds(s * rpw + k * C, C), :])

    pltpu.async_copy(q_hbm.at[srcv.at[0]], rows0, sem)
    pltpu.async_copy(q_hbm.at[srcv.at[1]], rows1, sem)
    plsc.subcore_barrier()

    pltpu.make_async_copy(q_hbm.at[srcv.at[0]], rows0, sem).wait()
    pltpu.sync_copy(rows0, acc_s.at[dstv.at[0]], add=True)
    pltpu.async_copy(q_hbm.at[srcv.at[2]], rows0, sem)

    def body(k, cr):
        a = 1 + 2 * k
        pltpu.make_async_copy(q_hbm.at[srcv.at[a]], rows1, sem).wait()
        pltpu.sync_copy(rows1, acc_s.at[dstv.at[a]], add=True)

        @pl.when(a + 2 < NCH)
        def _():
            pltpu.async_copy(q_hbm.at[srcv.at[a + 2]], rows1, sem)

        pltpu.make_async_copy(q_hbm.at[srcv.at[a + 1]], rows0, sem).wait()
        pltpu.sync_copy(rows0, acc_s.at[dstv.at[a + 1]], add=True)

        @pl.when(a + 3 < NCH)
        def _():
            pltpu.async_copy(q_hbm.at[srcv.at[a + 3]], rows0, sem)

        return cr

    lax.fori_loop(0, (NCH - 1) // 2, body, 0)
    plsc.subcore_barrier()
    pltpu.sync_copy(acc_s.at[pl.ds(s * rpw, rpw), :],
                    out_hbm.at[c, pl.ds(s * rpw, rpw), :])


@functools.partial(
    pl.kernel,
    out_type=jax.ShapeDtypeStruct((NW, NP), jnp.float32),
    mesh=_MESH,
    scratch_types=[
        pltpu.VMEM((NP,), jnp.float32),
        pltpu.VMEM((EPW,), jnp.int32),
        pltpu.VMEM((EPW,), jnp.int32),
        pltpu.VMEM((NP,), jnp.float32),
    ],
    compiler_params=pltpu.CompilerParams(
        needs_layout_passes=False, use_tc_tiling_on_sc=False),
)
def _sc_prop1(q_hbm, src_hbm, dst_hbm, out_hbm, qv, srcv, dstv, acc):
    wid = lax.axis_index("c") * NS + lax.axis_index("s")
    pltpu.sync_copy(q_hbm, qv)
    pltpu.sync_copy(src_hbm.at[wid], srcv)
    pltpu.sync_copy(dst_hbm.at[wid], dstv)

    zeros16 = jnp.zeros((16,), jnp.float32)

    def zbody(i, c):
        acc[pl.ds(i * 16, 16)] = zeros16
        return c

    lax.fori_loop(0, NP // 16, zbody, 0)

    def body(i, c):
        si = srcv[pl.ds(i * 16, 16)]
        di = dstv[pl.ds(i * 16, 16)]
        g = plsc.load_gather(qv, [si])
        plsc.addupdate_scatter(acc, [di], g)
        return c

    lax.fori_loop(0, EPW // 16, body, 0)
    pltpu.sync_copy(acc, out_hbm.at[wid])


_BRK = 1024
_G = NPK // _BRK


def _tc_dinv_body(degp_ref, o_ref):
    deg = jnp.sum(degp_ref[...], axis=0) + 1.0
    o_ref[...] = lax.rsqrt(deg).reshape(1, NP)


def _tc_dinv(deg_parts):
    return pl.pallas_call(
        _tc_dinv_body,
        out_shape=jax.ShapeDtypeStruct((1, NP), jnp.float32),
    )(deg_parts)


def _tc_l1_body(x_ref, w_ref, dinv_ref, o_ref):
    p = jnp.dot(x_ref[...], w_ref[...], preferred_element_type=jnp.float32)
    o_ref[...] = p * dinv_ref[...]


def _tc_l1(x2, w1blk, dinv_pk):
    return pl.pallas_call(
        _tc_l1_body,
        grid=(_G,),
        in_specs=[
            pl.BlockSpec((_BRK, 2 * D_IN), lambda g: (g, 0)),
            pl.BlockSpec((2 * D_IN, 128), lambda g: (0, 0)),
            pl.BlockSpec((_BRK, 128), lambda g: (g, 0)),
        ],
        out_specs=pl.BlockSpec((_BRK, 128), lambda g: (g, 0)),
        out_shape=jax.ShapeDtypeStruct((NPK, 128), jnp.float32),
    )(x2, w1blk, dinv_pk)


def _tc_mid_body(sp_ref, q_ref, dinv_ref, b_ref, w_ref, o_ref):
    s = sp_ref[0] + sp_ref[1]
    dinv = dinv_ref[...]
    h = jax.nn.relu(dinv * (s + q_ref[...]) + b_ref[...])
    o_ref[...] = jnp.dot(h, w_ref[...],
                         preferred_element_type=jnp.float32) * dinv


def _tc_mid(s_parts_pk, q_pk, dinv_pk, b_pk, wblk):
    return pl.pallas_call(
        _tc_mid_body,
        grid=(_G,),
        in_specs=[
            pl.BlockSpec((NC, _BRK, 128), lambda g: (0, g, 0)),
            pl.BlockSpec((_BRK, 128), lambda g: (g, 0)),
            pl.BlockSpec((_BRK, 128), lambda g: (g, 0)),
            pl.BlockSpec((1, 128), lambda g: (0, 0)),
            pl.BlockSpec((128, 128), lambda g: (0, 0)),
        ],
        out_specs=pl.BlockSpec((_BRK, 128), lambda g: (g, 0)),
        out_shape=jax.ShapeDtypeStruct((NPK, 128), jnp.float32),
    )(s_parts_pk, q_pk, dinv_pk, b_pk, wblk)


def _tc_final_body(sp_ref, q_ref, dinv_ref, b_ref, o_ref):
    s = jnp.sum(sp_ref[...], axis=0)
    o_ref[...] = dinv_ref[...] * (s.reshape(1, NP) + q_ref[...]) + b_ref[0, 0]


def _tc_final(s_parts, q3f, dinv1d, b3):
    return pl.pallas_call(
        _tc_final_body,
        out_shape=jax.ShapeDtypeStruct((1, NP), jnp.float32),
    )(s_parts, q3f, dinv1d, b3.reshape(1, 1))


def kernel(x, edge_index, W1, b1, W2, b2, W3, b3):
    src = edge_index[0]
    dst = edge_index[1]
    src3 = src.reshape(NW, NCH, C)
    dst3 = dst.reshape(NW, NCH, C)
    src2 = src.reshape(NW, EPW)
    dst2 = dst.reshape(NW, EPW)

    x_pad = jnp.pad(x, ((0, NP - N), (0, 0)))
    x2 = x_pad.reshape(NPK, 2 * D_IN)
    w1blk = (jnp.zeros((2 * D_IN, 128), jnp.float32)
             .at[:D_IN, :D_H].set(W1).at[D_IN:, D_H:].set(W1))
    w2blk = (jnp.zeros((128, 128), jnp.float32)
             .at[:D_H, :D_H].set(W2).at[D_H:, D_H:].set(W2))
    w3blk = (jnp.zeros((128, 128), jnp.float32)
             .at[:D_H, 0:1].set(W3).at[D_H:, D_H:D_H + 1].set(W3))
    b1pk = jnp.concatenate([b1, b1]).reshape(1, 128)
    b2pk = jnp.concatenate([b2, b2]).reshape(1, 128)

    deg_parts = _sc_degree(dst2)
    dinv1d = _tc_dinv(deg_parts)
    dinv_pk = jnp.repeat(dinv1d.reshape(NP), D_H).reshape(NPK, 128)

    q1 = _tc_l1(x2, w1blk, dinv_pk)
    s1 = _sc_prop(q1.reshape(NP, D_H), src3, dst3)
    q2 = _tc_mid(s1.reshape(NC, NPK, 128), q1, dinv_pk, b1pk, w2blk)
    s2 = _sc_prop(q2.reshape(NP, D_H), src3, dst3)
    q3 = _tc_mid(s2.reshape(NC, NPK, 128), q2, dinv_pk, b2pk, w3blk)
    q3f = q3.reshape(NPK, 2, D_H)[:, :, 0].reshape(NP)
    s3 = _sc_prop1(q3f, src2, dst2)
    out = _tc_final(s3, q3f.reshape(1, NP), dinv1d, b3)
    return out[0, :N].reshape(N, 1)


# speedup vs baseline: 51.7161x; 1.0506x over previous
import functools

import jax
import jax.numpy as jnp
from jax import lax
from jax.experimental import pallas as pl
from jax.experimental.pallas import tpu as pltpu
from jax.experimental.pallas import tpu_sc as plsc

N = 10000
E = 320000
D_IN = 128
D_H = 64
NP = 10240
NPK = NP // 2
NC = 2
NS = 16
NW = NC * NS
EPW = E // NW
C = 80
NCH = EPW // C

_MESH = plsc.VectorSubcoreMesh(
    core_axis_name="c", subcore_axis_name="s", num_cores=NC, num_subcores=NS)


@functools.partial(
    pl.kernel,
    out_type=jax.ShapeDtypeStruct((NW, NP), jnp.float32),
    mesh=_MESH,
    scratch_types=[
        pltpu.VMEM((EPW,), jnp.int32),
        pltpu.VMEM((NP,), jnp.float32),
    ],
    compiler_params=pltpu.CompilerParams(
        needs_layout_passes=False, use_tc_tiling_on_sc=False),
)
def _sc_degree(dst_hbm, out_hbm, dstv, acc):
    wid = lax.axis_index("c") * NS + lax.axis_index("s")
    pltpu.sync_copy(dst_hbm.at[wid], dstv)

    zeros16 = jnp.zeros((16,), jnp.float32)

    def zbody(i, c):
        acc[pl.ds(i * 16, 16)] = zeros16
        return c

    lax.fori_loop(0, NP // 16, zbody, 0)

    ones16 = jnp.ones((16,), jnp.float32)

    def body(i, c):
        idx = dstv[pl.ds(i * 16, 16)]
        plsc.addupdate_scatter(acc, [idx], ones16)
        return c

    lax.fori_loop(0, EPW // 16, body, 0)
    pltpu.sync_copy(acc, out_hbm.at[wid])


@functools.partial(
    pl.kernel,
    out_type=jax.ShapeDtypeStruct((NC, NP, D_H), jnp.float32),
    mesh=_MESH,
    scratch_types=[
        pltpu.VMEM((NCH, C), jnp.int32),
        pltpu.VMEM((NCH, C), jnp.int32),
        pltpu.VMEM((C, D_H), jnp.float32),
        pltpu.VMEM((C, D_H), jnp.float32),
        pltpu.VMEM_SHARED((NP, D_H), jnp.float32),
        pltpu.SemaphoreType.DMA,
    ],
    compiler_params=pltpu.CompilerParams(use_tc_tiling_on_sc=False),
)
def _sc_prop(q_hbm, src_hbm, dst_hbm, out_hbm, srcv, dstv, rows0, rows1,
             acc_s, sem):
    c = lax.axis_index("c")
    s = lax.axis_index("s")
    wid = c * NS + s
    pltpu.sync_copy(src_hbm.at[wid], srcv)
    pltpu.sync_copy(dst_hbm.at[wid], dstv)

    zeros16 = jnp.zeros((16,), jnp.float32)

    def zbody(i, cr):
        rows0[i >> 2, pl.ds((i & 3) * 16, 16)] = zeros16
        return cr

    lax.fori_loop(0, C * D_H // 16, zbody, 0)
    rpw = NP // NS
    for k in range(rpw // C):
        pltpu.sync_copy(rows0, acc_s.at[pl.ds(s * rpw + k * C, C), :])

    pltpu.async_copy(q_hbm.at[srcv.at[0]], rows0, sem)
    pltpu.async_copy(q_hbm.at[srcv.at[1]], rows1, sem)
    plsc.subcore_barrier()

    pltpu.make_async_copy(q_hbm.at[srcv.at[0]], rows0, sem).wait()
    pltpu.sync_copy(rows0, acc_s.at[dstv.at[0]], add=True)
    pltpu.async_copy(q_hbm.at[srcv.at[2]], rows0, sem)

    def body(k, cr):
        a = 1 + 2 * k
        pltpu.make_async_copy(q_hbm.at[srcv.at[a]], rows1, sem).wait()
        pltpu.sync_copy(rows1, acc_s.at[dstv.at[a]], add=True)

        @pl.when(a + 2 < NCH)
        def _():
            pltpu.async_copy(q_hbm.at[srcv.at[a + 2]], rows1, sem)

        pltpu.make_async_copy(q_hbm.at[srcv.at[a + 1]], rows0, sem).wait()
        pltpu.sync_copy(rows0, acc_s.at[dstv.at[a + 1]], add=True)

        @pl.when(a + 3 < NCH)
        def _():
            pltpu.async_copy(q_hbm.at[srcv.at[a + 3]], rows0, sem)

        return cr

    lax.fori_loop(0, (NCH - 1) // 2, body, 0)
    plsc.subcore_barrier()
    pltpu.sync_copy(acc_s.at[pl.ds(s * rpw, rpw), :],
                    out_hbm.at[c, pl.ds(s * rpw, rpw), :])


@functools.partial(
    pl.kernel,
    out_type=jax.ShapeDtypeStruct((NW, NP), jnp.float32),
    mesh=_MESH,
    scratch_types=[
        pltpu.VMEM((NP,), jnp.float32),
        pltpu.VMEM((EPW,), jnp.int32),
        pltpu.VMEM((EPW,), jnp.int32),
        pltpu.VMEM((NP,), jnp.float32),
    ],
    compiler_params=pltpu.CompilerParams(
        needs_layout_passes=False, use_tc_tiling_on_sc=False),
)
def _sc_prop1(q_hbm, src_hbm, dst_hbm, out_hbm, qv, srcv, dstv, acc):
    wid = lax.axis_index("c") * NS + lax.axis_index("s")
    pltpu.sync_copy(q_hbm, qv)
    pltpu.sync_copy(src_hbm.at[wid], srcv)
    pltpu.sync_copy(dst_hbm.at[wid], dstv)

    zeros16 = jnp.zeros((16,), jnp.float32)

    def zbody(i, c):
        acc[pl.ds(i * 16, 16)] = zeros16
        return c

    lax.fori_loop(0, NP // 16, zbody, 0)

    def body(i, c):
        si = srcv[pl.ds(i * 16, 16)]
        di = dstv[pl.ds(i * 16, 16)]
        g = plsc.load_gather(qv, [si])
        plsc.addupdate_scatter(acc, [di], g)
        return c

    lax.fori_loop(0, EPW // 16, body, 0)
    pltpu.sync_copy(acc, out_hbm.at[wid])


_BRK = 1024
_G = NPK // _BRK


def _tc_edges_body(ei_ref, src_ref, dst_ref):
    src_ref[...] = ei_ref[0]
    dst_ref[...] = ei_ref[1]


def _tc_edges(edge_index):
    return pl.pallas_call(
        _tc_edges_body,
        out_shape=[jax.ShapeDtypeStruct((E,), jnp.int32),
                   jax.ShapeDtypeStruct((E,), jnp.int32)],
    )(edge_index)


def _tc_dinv_body(degp_ref, o_ref):
    deg = jnp.sum(degp_ref[...], axis=0) + 1.0
    o_ref[...] = lax.rsqrt(deg).reshape(1, NP)


def _tc_dinv(deg_parts):
    return pl.pallas_call(
        _tc_dinv_body,
        out_shape=jax.ShapeDtypeStruct((1, NP), jnp.float32),
    )(deg_parts)


def _tc_l1_body(x_ref, w_ref, dinv_ref, o_ref):
    p = jnp.dot(x_ref[...], w_ref[...], preferred_element_type=jnp.float32)
    o_ref[...] = p * dinv_ref[...]


def _tc_l1(x2, w1blk, dinv_pk):
    return pl.pallas_call(
        _tc_l1_body,
        grid=(_G,),
        in_specs=[
            pl.BlockSpec((_BRK, 2 * D_IN), lambda g: (g, 0)),
            pl.BlockSpec((2 * D_IN, 128), lambda g: (0, 0)),
            pl.BlockSpec((_BRK, 128), lambda g: (g, 0)),
        ],
        out_specs=pl.BlockSpec((_BRK, 128), lambda g: (g, 0)),
        out_shape=jax.ShapeDtypeStruct((NPK, 128), jnp.float32),
    )(x2, w1blk, dinv_pk)


def _tc_mid_body(sp_ref, q_ref, dinv_ref, b_ref, w_ref, o_ref):
    s = sp_ref[0] + sp_ref[1]
    dinv = dinv_ref[...]
    h = jax.nn.relu(dinv * (s + q_ref[...]) + b_ref[...])
    o_ref[...] = jnp.dot(h, w_ref[...],
                         preferred_element_type=jnp.float32) * dinv


def _tc_mid(s_parts_pk, q_pk, dinv_pk, b_pk, wblk):
    return pl.pallas_call(
        _tc_mid_body,
        grid=(_G,),
        in_specs=[
            pl.BlockSpec((NC, _BRK, 128), lambda g: (0, g, 0)),
            pl.BlockSpec((_BRK, 128), lambda g: (g, 0)),
            pl.BlockSpec((_BRK, 128), lambda g: (g, 0)),
            pl.BlockSpec((1, 128), lambda g: (0, 0)),
            pl.BlockSpec((128, 128), lambda g: (0, 0)),
        ],
        out_specs=pl.BlockSpec((_BRK, 128), lambda g: (g, 0)),
        out_shape=jax.ShapeDtypeStruct((NPK, 128), jnp.float32),
    )(s_parts_pk, q_pk, dinv_pk, b_pk, wblk)


def _tc_final_body(sp_ref, q_ref, dinv_ref, b_ref, o_ref):
    s = jnp.sum(sp_ref[...], axis=0)
    o_ref[...] = dinv_ref[...] * (s.reshape(1, NP) + q_ref[...]) + b_ref[0, 0]


def _tc_final(s_parts, q3f, dinv1d, b3):
    return pl.pallas_call(
        _tc_final_body,
        out_shape=jax.ShapeDtypeStruct((1, NP), jnp.float32),
    )(s_parts, q3f, dinv1d, b3.reshape(1, 1))


def kernel(x, edge_index, W1, b1, W2, b2, W3, b3):
    src, dst = _tc_edges(edge_index)
    src3 = src.reshape(NW, NCH, C)
    dst3 = dst.reshape(NW, NCH, C)
    src2 = src.reshape(NW, EPW)
    dst2 = dst.reshape(NW, EPW)

    x_pad = jnp.pad(x, ((0, NP - N), (0, 0)))
    x2 = x_pad.reshape(NPK, 2 * D_IN)
    w1blk = (jnp.zeros((2 * D_IN, 128), jnp.float32)
             .at[:D_IN, :D_H].set(W1).at[D_IN:, D_H:].set(W1))
    w2blk = (jnp.zeros((128, 128), jnp.float32)
             .at[:D_H, :D_H].set(W2).at[D_H:, D_H:].set(W2))
    w3blk = (jnp.zeros((128, 128), jnp.float32)
             .at[:D_H, 0:1].set(W3).at[D_H:, D_H:D_H + 1].set(W3))
    b1pk = jnp.concatenate([b1, b1]).reshape(1, 128)
    b2pk = jnp.concatenate([b2, b2]).reshape(1, 128)

    deg_parts = _sc_degree(dst2)
    dinv1d = _tc_dinv(deg_parts)
    dinv_pk = jnp.repeat(dinv1d.reshape(NP), D_H).reshape(NPK, 128)

    q1 = _tc_l1(x2, w1blk, dinv_pk)
    s1 = _sc_prop(q1.reshape(NP, D_H), src3, dst3)
    q2 = _tc_mid(s1.reshape(NC, NPK, 128), q1, dinv_pk, b1pk, w2blk)
    s2 = _sc_prop(q2.reshape(NP, D_H), src3, dst3)
    q3 = _tc_mid(s2.reshape(NC, NPK, 128), q2, dinv_pk, b2pk, w3blk)
    q3f = q3.reshape(NPK, 2, D_H)[:, :, 0].reshape(NP)
    s3 = _sc_prop1(q3f, src2, dst2)
    out = _tc_final(s3, q3f.reshape(1, NP), dinv1d, b3)
    return out[0, :N].reshape(N, 1)


# speedup vs baseline: 57.9844x; 1.1212x over previous
import functools

import jax
import jax.numpy as jnp
from jax import lax
from jax.experimental import pallas as pl
from jax.experimental.pallas import tpu as pltpu
from jax.experimental.pallas import tpu_sc as plsc

N = 10000
E = 320000
D_IN = 128
D_H = 64
NP = 10240
NPK = NP // 2
NC = 2
NS = 16
NW = NC * NS
EPW = E // NW
C = 80
NCH = EPW // C

_MESH = plsc.VectorSubcoreMesh(
    core_axis_name="c", subcore_axis_name="s", num_cores=NC, num_subcores=NS)


@functools.partial(
    pl.kernel,
    out_type=jax.ShapeDtypeStruct((NW, NP), jnp.float32),
    mesh=_MESH,
    scratch_types=[
        pltpu.VMEM((EPW,), jnp.int32),
        pltpu.VMEM((NP,), jnp.float32),
    ],
    compiler_params=pltpu.CompilerParams(
        needs_layout_passes=False, use_tc_tiling_on_sc=False),
)
def _sc_degree(dst_hbm, out_hbm, dstv, acc):
    wid = lax.axis_index("c") * NS + lax.axis_index("s")
    pltpu.sync_copy(dst_hbm.at[wid], dstv)

    zeros16 = jnp.zeros((16,), jnp.float32)

    def zbody(i, c):
        acc[pl.ds(i * 16, 16)] = zeros16
        return c

    lax.fori_loop(0, NP // 16, zbody, 0)

    ones16 = jnp.ones((16,), jnp.float32)

    def body(i, c):
        idx = dstv[pl.ds(i * 16, 16)]
        plsc.addupdate_scatter(acc, [idx], ones16)
        return c

    lax.fori_loop(0, EPW // 16, body, 0)
    pltpu.sync_copy(acc, out_hbm.at[wid])


@functools.partial(
    pl.kernel,
    out_type=jax.ShapeDtypeStruct((NC, NP, D_H), jnp.float32),
    mesh=_MESH,
    scratch_types=[
        pltpu.VMEM((NCH, C), jnp.int32),
        pltpu.VMEM((NCH, C), jnp.int32),
        pltpu.VMEM((C, D_H), jnp.float32),
        pltpu.VMEM((C, D_H), jnp.float32),
        pltpu.VMEM((C, D_H), jnp.float32),
        pltpu.VMEM_SHARED((NP, D_H), jnp.float32),
        pltpu.SemaphoreType.DMA,
        pltpu.SemaphoreType.DMA,
    ],
    compiler_params=pltpu.CompilerParams(use_tc_tiling_on_sc=False),
)
def _sc_prop(q_hbm, src_hbm, dst_hbm, out_hbm, srcv, dstv, rows0, rows1,
             rows2, acc_s, gsem, ssem):
    c = lax.axis_index("c")
    s = lax.axis_index("s")
    wid = c * NS + s
    pltpu.sync_copy(src_hbm.at[wid], srcv)
    pltpu.sync_copy(dst_hbm.at[wid], dstv)

    zeros16 = jnp.zeros((16,), jnp.float32)

    def zbody(i, cr):
        rows0[i >> 2, pl.ds((i & 3) * 16, 16)] = zeros16
        return cr

    lax.fori_loop(0, C * D_H // 16, zbody, 0)
    rpw = NP // NS
    for k in range(rpw // C):
        pltpu.sync_copy(rows0, acc_s.at[pl.ds(s * rpw + k * C, C), :])

    bufs = (rows0, rows1, rows2)

    def wait_g(j, buf):
        pltpu.make_async_copy(q_hbm.at[srcv.at[j]], buf, gsem).wait()

    def start_g(j, buf):
        pltpu.async_copy(q_hbm.at[srcv.at[j]], buf, gsem)

    def start_s(j, buf):
        pltpu.async_copy(buf, acc_s.at[dstv.at[j]], ssem, add=True)

    def wait_s(j, buf):
        pltpu.make_async_copy(buf, acc_s.at[dstv.at[j]], ssem).wait()

    start_g(0, rows0)
    start_g(1, rows1)
    plsc.subcore_barrier()

    wait_g(0, rows0)
    start_s(0, rows0)
    start_g(2, rows2)
    wait_g(1, rows1)
    start_s(1, rows1)
    wait_s(0, rows0)
    start_g(3, rows0)

    def body(k, cr):
        j0 = 2 + 3 * k
        for i in range(3):
            j = j0 + i
            bj = bufs[(2 + i) % 3]
            bn = bufs[(2 + i + 2) % 3]
            wait_g(j, bj)
            start_s(j, bj)
            wait_s(j - 1, bufs[(2 + i + 2) % 3])
            @pl.when(j + 2 < NCH)
            def _():
                start_g(j + 2, bn)
        return cr

    lax.fori_loop(0, (NCH - 2) // 3, body, 0)
    wait_s(NCH - 1, bufs[(NCH - 1) % 3])
    plsc.subcore_barrier()
    pltpu.sync_copy(acc_s.at[pl.ds(s * rpw, rpw), :],
                    out_hbm.at[c, pl.ds(s * rpw, rpw), :])


@functools.partial(
    pl.kernel,
    out_type=jax.ShapeDtypeStruct((NW, NP), jnp.float32),
    mesh=_MESH,
    scratch_types=[
        pltpu.VMEM((NP,), jnp.float32),
        pltpu.VMEM((EPW,), jnp.int32),
        pltpu.VMEM((EPW,), jnp.int32),
        pltpu.VMEM((NP,), jnp.float32),
    ],
    compiler_params=pltpu.CompilerParams(
        needs_layout_passes=False, use_tc_tiling_on_sc=False),
)
def _sc_prop1(q_hbm, src_hbm, dst_hbm, out_hbm, qv, srcv, dstv, acc):
    wid = lax.axis_index("c") * NS + lax.axis_index("s")
    pltpu.sync_copy(q_hbm, qv)
    pltpu.sync_copy(src_hbm.at[wid], srcv)
    pltpu.sync_copy(dst_hbm.at[wid], dstv)

    zeros16 = jnp.zeros((16,), jnp.float32)

    def zbody(i, c):
        acc[pl.ds(i * 16, 16)] = zeros16
        return c

    lax.fori_loop(0, NP // 16, zbody, 0)

    def body(i, c):
        si = srcv[pl.ds(i * 16, 16)]
        di = dstv[pl.ds(i * 16, 16)]
        g = plsc.load_gather(qv, [si])
        plsc.addupdate_scatter(acc, [di], g)
        return c

    lax.fori_loop(0, EPW // 16, body, 0)
    pltpu.sync_copy(acc, out_hbm.at[wid])


_BRK = 1024
_G = NPK // _BRK


def _tc_edges_body(ei_ref, src_ref, dst_ref):
    src_ref[...] = ei_ref[0]
    dst_ref[...] = ei_ref[1]


def _tc_edges(edge_index):
    return pl.pallas_call(
        _tc_edges_body,
        out_shape=[jax.ShapeDtypeStruct((E,), jnp.int32),
                   jax.ShapeDtypeStruct((E,), jnp.int32)],
    )(edge_index)


def _tc_dinv_body(degp_ref, o_ref):
    deg = jnp.sum(degp_ref[...], axis=0) + 1.0
    o_ref[...] = lax.rsqrt(deg).reshape(1, NP)


def _tc_dinv(deg_parts):
    return pl.pallas_call(
        _tc_dinv_body,
        out_shape=jax.ShapeDtypeStruct((1, NP), jnp.float32),
    )(deg_parts)


def _tc_l1_body(x_ref, w_ref, dinv_ref, o_ref):
    p = jnp.dot(x_ref[...], w_ref[...], preferred_element_type=jnp.float32)
    o_ref[...] = p * dinv_ref[...]


def _tc_l1(x2, w1blk, dinv_pk):
    return pl.pallas_call(
        _tc_l1_body,
        grid=(_G,),
        in_specs=[
            pl.BlockSpec((_BRK, 2 * D_IN), lambda g: (g, 0)),
            pl.BlockSpec((2 * D_IN, 128), lambda g: (0, 0)),
            pl.BlockSpec((_BRK, 128), lambda g: (g, 0)),
        ],
        out_specs=pl.BlockSpec((_BRK, 128), lambda g: (g, 0)),
        out_shape=jax.ShapeDtypeStruct((NPK, 128), jnp.float32),
    )(x2, w1blk, dinv_pk)


def _tc_mid_body(sp_ref, q_ref, dinv_ref, b_ref, w_ref, o_ref):
    s = sp_ref[0] + sp_ref[1]
    dinv = dinv_ref[...]
    h = jax.nn.relu(dinv * (s + q_ref[...]) + b_ref[...])
    o_ref[...] = jnp.dot(h, w_ref[...],
                         preferred_element_type=jnp.float32) * dinv


def _tc_mid(s_parts_pk, q_pk, dinv_pk, b_pk, wblk):
    return pl.pallas_call(
        _tc_mid_body,
        grid=(_G,),
        in_specs=[
            pl.BlockSpec((NC, _BRK, 128), lambda g: (0, g, 0)),
            pl.BlockSpec((_BRK, 128), lambda g: (g, 0)),
            pl.BlockSpec((_BRK, 128), lambda g: (g, 0)),
            pl.BlockSpec((1, 128), lambda g: (0, 0)),
            pl.BlockSpec((128, 128), lambda g: (0, 0)),
        ],
        out_specs=pl.BlockSpec((_BRK, 128), lambda g: (g, 0)),
        out_shape=jax.ShapeDtypeStruct((NPK, 128), jnp.float32),
    )(s_parts_pk, q_pk, dinv_pk, b_pk, wblk)


def _tc_final_body(sp_ref, q_ref, dinv_ref, b_ref, o_ref):
    s = jnp.sum(sp_ref[...], axis=0)
    o_ref[...] = dinv_ref[...] * (s.reshape(1, NP) + q_ref[...]) + b_ref[0, 0]


def _tc_final(s_parts, q3f, dinv1d, b3):
    return pl.pallas_call(
        _tc_final_body,
        out_shape=jax.ShapeDtypeStruct((1, NP), jnp.float32),
    )(s_parts, q3f, dinv1d, b3.reshape(1, 1))


def kernel(x, edge_index, W1, b1, W2, b2, W3, b3):
    src, dst = _tc_edges(edge_index)
    src3 = src.reshape(NW, NCH, C)
    dst3 = dst.reshape(NW, NCH, C)
    src2 = src.reshape(NW, EPW)
    dst2 = dst.reshape(NW, EPW)

    x_pad = jnp.pad(x, ((0, NP - N), (0, 0)))
    x2 = x_pad.reshape(NPK, 2 * D_IN)
    w1blk = (jnp.zeros((2 * D_IN, 128), jnp.float32)
             .at[:D_IN, :D_H].set(W1).at[D_IN:, D_H:].set(W1))
    w2blk = (jnp.zeros((128, 128), jnp.float32)
             .at[:D_H, :D_H].set(W2).at[D_H:, D_H:].set(W2))
    w3blk = (jnp.zeros((128, 128), jnp.float32)
             .at[:D_H, 0:1].set(W3).at[D_H:, D_H:D_H + 1].set(W3))
    b1pk = jnp.concatenate([b1, b1]).reshape(1, 128)
    b2pk = jnp.concatenate([b2, b2]).reshape(1, 128)

    deg_parts = _sc_degree(dst2)
    dinv1d = _tc_dinv(deg_parts)
    dinv_pk = jnp.repeat(dinv1d.reshape(NP), D_H).reshape(NPK, 128)

    q1 = _tc_l1(x2, w1blk, dinv_pk)
    s1 = _sc_prop(q1.reshape(NP, D_H), src3, dst3)
    q2 = _tc_mid(s1.reshape(NC, NPK, 128), q1, dinv_pk, b1pk, w2blk)
    s2 = _sc_prop(q2.reshape(NP, D_H), src3, dst3)
    q3 = _tc_mid(s2.reshape(NC, NPK, 128), q2, dinv_pk, b2pk, w3blk)
    q3f = q3.reshape(NPK, 2, D_H)[:, :, 0].reshape(NP)
    s3 = _sc_prop1(q3f, src2, dst2)
    out = _tc_final(s3, q3f.reshape(1, NP), dinv1d, b3)
    return out[0, :N].reshape(N, 1)


# speedup vs baseline: 61.2941x; 1.0571x over previous
import functools

import jax
import jax.numpy as jnp
from jax import lax
from jax.experimental import pallas as pl
from jax.experimental.pallas import tpu as pltpu
from jax.experimental.pallas import tpu_sc as plsc

N = 10000
E = 320000
D_IN = 128
D_H = 64
NP = 10240
NPK = NP // 2
NC = 2
NS = 16
NW = NC * NS
EPW = E // NW
C = 125
NCH = EPW // C

_MESH = plsc.VectorSubcoreMesh(
    core_axis_name="c", subcore_axis_name="s", num_cores=NC, num_subcores=NS)


@functools.partial(
    pl.kernel,
    out_type=jax.ShapeDtypeStruct((NW, NP), jnp.float32),
    mesh=_MESH,
    scratch_types=[
        pltpu.VMEM((EPW,), jnp.int32),
        pltpu.VMEM((NP,), jnp.float32),
    ],
    compiler_params=pltpu.CompilerParams(
        needs_layout_passes=False, use_tc_tiling_on_sc=False),
)
def _sc_degree(dst_hbm, out_hbm, dstv, acc):
    wid = lax.axis_index("c") * NS + lax.axis_index("s")
    pltpu.sync_copy(dst_hbm.at[wid], dstv)

    zeros16 = jnp.zeros((16,), jnp.float32)

    def zbody(i, c):
        acc[pl.ds(i * 16, 16)] = zeros16
        return c

    lax.fori_loop(0, NP // 16, zbody, 0)

    ones16 = jnp.ones((16,), jnp.float32)

    def body(i, c):
        idx = dstv[pl.ds(i * 16, 16)]
        plsc.addupdate_scatter(acc, [idx], ones16)
        return c

    lax.fori_loop(0, EPW // 16, body, 0)
    pltpu.sync_copy(acc, out_hbm.at[wid])


@functools.partial(
    pl.kernel,
    out_type=jax.ShapeDtypeStruct((NC, NP, D_H), jnp.float32),
    mesh=_MESH,
    scratch_types=[
        pltpu.VMEM((NCH, C), jnp.int32),
        pltpu.VMEM((NCH, C), jnp.int32),
        pltpu.VMEM((C, D_H), jnp.float32),
        pltpu.VMEM((C, D_H), jnp.float32),
        pltpu.VMEM((C, D_H), jnp.float32),
        pltpu.VMEM_SHARED((NP, D_H), jnp.float32),
        pltpu.SemaphoreType.DMA,
        pltpu.SemaphoreType.DMA,
    ],
    compiler_params=pltpu.CompilerParams(use_tc_tiling_on_sc=False),
)
def _sc_prop(q_hbm, src_hbm, dst_hbm, out_hbm, srcv, dstv, rows0, rows1,
             rows2, acc_s, gsem, ssem):
    c = lax.axis_index("c")
    s = lax.axis_index("s")
    wid = c * NS + s
    pltpu.sync_copy(src_hbm.at[wid], srcv)
    pltpu.sync_copy(dst_hbm.at[wid], dstv)

    zeros16 = jnp.zeros((16,), jnp.float32)

    def zbody(i, cr):
        rows0[i >> 2, pl.ds((i & 3) * 16, 16)] = zeros16
        return cr

    lax.fori_loop(0, C * D_H // 16, zbody, 0)
    rpw = NP // NS
    for k in range(rpw // C):
        pltpu.sync_copy(rows0, acc_s.at[pl.ds(s * rpw + k * C, C), :])

    bufs = (rows0, rows1, rows2)

    def wait_g(j, buf):
        pltpu.make_async_copy(q_hbm.at[srcv.at[j]], buf, gsem).wait()

    def start_g(j, buf):
        pltpu.async_copy(q_hbm.at[srcv.at[j]], buf, gsem)

    def start_s(j, buf):
        pltpu.async_copy(buf, acc_s.at[dstv.at[j]], ssem, add=True)

    def wait_s(j, buf):
        pltpu.make_async_copy(buf, acc_s.at[dstv.at[j]], ssem).wait()

    start_g(0, rows0)
    start_g(1, rows1)
    plsc.subcore_barrier()

    wait_g(0, rows0)
    start_s(0, rows0)
    start_g(2, rows2)
    wait_g(1, rows1)
    start_s(1, rows1)
    wait_s(0, rows0)
    start_g(3, rows0)

    def body(k, cr):
        j0 = 2 + 3 * k
        for i in range(3):
            j = j0 + i
            bj = bufs[(2 + i) % 3]
            bn = bufs[(2 + i + 2) % 3]
            wait_g(j, bj)
            start_s(j, bj)
            wait_s(j - 1, bufs[(2 + i + 2) % 3])
            @pl.when(j + 2 < NCH)
            def _():
                start_g(j + 2, bn)
        return cr

    lax.fori_loop(0, (NCH - 2) // 3, body, 0)
    wait_s(NCH - 1, bufs[(NCH - 1) % 3])
    plsc.subcore_barrier()
    pltpu.sync_copy(acc_s.at[pl.ds(s * rpw, rpw), :],
                    out_hbm.at[c, pl.ds(s * rpw, rpw), :])


@functools.partial(
    pl.kernel,
    out_type=jax.ShapeDtypeStruct((NW, NP), jnp.float32),
    mesh=_MESH,
    scratch_types=[
        pltpu.VMEM((NP,), jnp.float32),
        pltpu.VMEM((EPW,), jnp.int32),
        pltpu.VMEM((EPW,), jnp.int32),
        pltpu.VMEM((NP,), jnp.float32),
    ],
    compiler_params=pltpu.CompilerParams(
        needs_layout_passes=False, use_tc_tiling_on_sc=False),
)
def _sc_prop1(q_hbm, src_hbm, dst_hbm, out_hbm, qv, srcv, dstv, acc):
    wid = lax.axis_index("c") * NS + lax.axis_index("s")
    pltpu.sync_copy(q_hbm, qv)
    pltpu.sync_copy(src_hbm.at[wid], srcv)
    pltpu.sync_copy(dst_hbm.at[wid], dstv)

    zeros16 = jnp.zeros((16,), jnp.float32)

    def zbody(i, c):
        acc[pl.ds(i * 16, 16)] = zeros16
        return c

    lax.fori_loop(0, NP // 16, zbody, 0)

    def body(i, c):
        si = srcv[pl.ds(i * 16, 16)]
        di = dstv[pl.ds(i * 16, 16)]
        g = plsc.load_gather(qv, [si])
        plsc.addupdate_scatter(acc, [di], g)
        return c

    lax.fori_loop(0, EPW // 16, body, 0)
    pltpu.sync_copy(acc, out_hbm.at[wid])


_BRK = 1024
_G = NPK // _BRK


def _tc_edges_body(ei_ref, src_ref, dst_ref):
    src_ref[...] = ei_ref[0]
    dst_ref[...] = ei_ref[1]


def _tc_edges(edge_index):
    return pl.pallas_call(
        _tc_edges_body,
        out_shape=[jax.ShapeDtypeStruct((E,), jnp.int32),
                   jax.ShapeDtypeStruct((E,), jnp.int32)],
    )(edge_index)


def _tc_dinv_body(degp_ref, o_ref):
    deg = jnp.sum(degp_ref[...], axis=0) + 1.0
    o_ref[...] = lax.rsqrt(deg).reshape(1, NP)


def _tc_dinv(deg_parts):
    return pl.pallas_call(
        _tc_dinv_body,
        out_shape=jax.ShapeDtypeStruct((1, NP), jnp.float32),
    )(deg_parts)


def _tc_l1_body(x_ref, w_ref, dinv_ref, o_ref):
    p = jnp.dot(x_ref[...], w_ref[...], preferred_element_type=jnp.float32)
    o_ref[...] = p * dinv_ref[...]


def _tc_l1(x2, w1blk, dinv_pk):
    return pl.pallas_call(
        _tc_l1_body,
        grid=(_G,),
        in_specs=[
            pl.BlockSpec((_BRK, 2 * D_IN), lambda g: (g, 0)),
            pl.BlockSpec((2 * D_IN, 128), lambda g: (0, 0)),
            pl.BlockSpec((_BRK, 128), lambda g: (g, 0)),
        ],
        out_specs=pl.BlockSpec((_BRK, 128), lambda g: (g, 0)),
        out_shape=jax.ShapeDtypeStruct((NPK, 128), jnp.float32),
    )(x2, w1blk, dinv_pk)


def _tc_mid_body(sp_ref, q_ref, dinv_ref, b_ref, w_ref, o_ref):
    s = sp_ref[0] + sp_ref[1]
    dinv = dinv_ref[...]
    h = jax.nn.relu(dinv * (s + q_ref[...]) + b_ref[...])
    o_ref[...] = jnp.dot(h, w_ref[...],
                         preferred_element_type=jnp.float32) * dinv


def _tc_mid(s_parts_pk, q_pk, dinv_pk, b_pk, wblk):
    return pl.pallas_call(
        _tc_mid_body,
        grid=(_G,),
        in_specs=[
            pl.BlockSpec((NC, _BRK, 128), lambda g: (0, g, 0)),
            pl.BlockSpec((_BRK, 128), lambda g: (g, 0)),
            pl.BlockSpec((_BRK, 128), lambda g: (g, 0)),
            pl.BlockSpec((1, 128), lambda g: (0, 0)),
            pl.BlockSpec((128, 128), lambda g: (0, 0)),
        ],
        out_specs=pl.BlockSpec((_BRK, 128), lambda g: (g, 0)),
        out_shape=jax.ShapeDtypeStruct((NPK, 128), jnp.float32),
    )(s_parts_pk, q_pk, dinv_pk, b_pk, wblk)


def _tc_final_body(sp_ref, q_ref, dinv_ref, b_ref, o_ref):
    s = jnp.sum(sp_ref[...], axis=0)
    o_ref[...] = dinv_ref[...] * (s.reshape(1, NP) + q_ref[...]) + b_ref[0, 0]


def _tc_final(s_parts, q3f, dinv1d, b3):
    return pl.pallas_call(
        _tc_final_body,
        out_shape=jax.ShapeDtypeStruct((1, NP), jnp.float32),
    )(s_parts, q3f, dinv1d, b3.reshape(1, 1))


def kernel(x, edge_index, W1, b1, W2, b2, W3, b3):
    src, dst = _tc_edges(edge_index)
    src3 = src.reshape(NW, NCH, C)
    dst3 = dst.reshape(NW, NCH, C)
    src2 = src.reshape(NW, EPW)
    dst2 = dst.reshape(NW, EPW)

    x_pad = jnp.pad(x, ((0, NP - N), (0, 0)))
    x2 = x_pad.reshape(NPK, 2 * D_IN)
    w1blk = (jnp.zeros((2 * D_IN, 128), jnp.float32)
             .at[:D_IN, :D_H].set(W1).at[D_IN:, D_H:].set(W1))
    w2blk = (jnp.zeros((128, 128), jnp.float32)
             .at[:D_H, :D_H].set(W2).at[D_H:, D_H:].set(W2))
    w3blk = (jnp.zeros((128, 128), jnp.float32)
             .at[:D_H, 0:1].set(W3).at[D_H:, D_H:D_H + 1].set(W3))
    b1pk = jnp.concatenate([b1, b1]).reshape(1, 128)
    b2pk = jnp.concatenate([b2, b2]).reshape(1, 128)

    deg_parts = _sc_degree(dst2)
    dinv1d = _tc_dinv(deg_parts)
    dinv_pk = jnp.repeat(dinv1d.reshape(NP), D_H).reshape(NPK, 128)

    q1 = _tc_l1(x2, w1blk, dinv_pk)
    s1 = _sc_prop(q1.reshape(NP, D_H), src3, dst3)
    q2 = _tc_mid(s1.reshape(NC, NPK, 128), q1, dinv_pk, b1pk, w2blk)
    s2 = _sc_prop(q2.reshape(NP, D_H), src3, dst3)
    q3 = _tc_mid(s2.reshape(NC, NPK, 128), q2, dinv_pk, b2pk, w3blk)
    q3f = q3.reshape(NPK, 2, D_H)[:, :, 0].reshape(NP)
    s3 = _sc_prop1(q3f, src2, dst2)
    out = _tc_final(s3, q3f.reshape(1, NP), dinv1d, b3)
    return out[0, :N].reshape(N, 1)
